# baseline (speedup 1.0000x reference)
import jax
import jax.numpy as jnp
from jax import lax
from jax.experimental import pallas as pl
from jax.experimental.pallas import tpu as pltpu
from jax.experimental.pallas import tpu_sc as plsc


def _sc_body(yt_hbm, yp_hbm, tvec_hbm, out_hbm, idx_s, yt_s, yp_s, out_s):
    pltpu.sync_copy(tvec_hbm, idx_s)
    t = idx_s[0]
    base = pl.multiple_of((t // 8) * 8, 8)
    r = t - base
    pltpu.sync_copy(yt_hbm.at[pl.ds(base, 8), :], yt_s)
    pltpu.sync_copy(yp_hbm.at[pl.ds(base, 8), :], yp_s)
    d = yp_s[r, 0] - yt_s[r, 0]
    out_s[0] = -(d * d)
    pltpu.sync_copy(out_s, out_hbm)


def _sc_call(yt_t, yp_t, tvec):
    mesh = plsc.ScalarSubcoreMesh(axis_name="c", num_cores=1)
    return pl.kernel(
        _sc_body,
        out_type=jax.ShapeDtypeStruct((8,), jnp.float32),
        mesh=mesh,
        compiler_params=pltpu.CompilerParams(
            needs_layout_passes=False, use_tc_tiling_on_sc=True
        ),
        scratch_types=[
            pltpu.SMEM((1,), jnp.int32),
            pltpu.SMEM((8, 128), jnp.float32),
            pltpu.SMEM((8, 128), jnp.float32),
            pltpu.SMEM((8,), jnp.float32),
        ],
    )(yt_t, yp_t, tvec)


def kernel(y_true, y_pred, target):
    t = jnp.asarray(target, jnp.int32)
    tvec = jnp.reshape(t, (1,))
    out = _sc_call(y_true.T, y_pred.T, tvec)
    return out[0]


# speedup vs baseline: 1.0454x; 1.0454x over previous
import jax
import jax.numpy as jnp
from jax import lax
from jax.experimental import pallas as pl
from jax.experimental.pallas import tpu as pltpu
from jax.experimental.pallas import tpu_sc as plsc


def _sc_body(yt_hbm, yp_hbm, tvec_hbm, out_hbm, idx_s, yt_s, yp_s, out_s, sem):
    pltpu.sync_copy(tvec_hbm, idx_s)
    t = idx_s[0]
    cp_t = pltpu.make_async_copy(yt_hbm.at[pl.ds(t, 1), :], yt_s, sem)
    cp_p = pltpu.make_async_copy(yp_hbm.at[pl.ds(t, 1), :], yp_s, sem)
    cp_t.start()
    cp_p.start()
    cp_t.wait()
    cp_p.wait()
    d = yp_s[0, 0] - yt_s[0, 0]
    out_s[0] = -(d * d)
    pltpu.sync_copy(out_s, out_hbm)


def _sc_call(yt_t, yp_t, tvec):
    mesh = plsc.ScalarSubcoreMesh(axis_name="c", num_cores=1)
    return pl.kernel(
        _sc_body,
        out_type=jax.ShapeDtypeStruct((8,), jnp.float32),
        mesh=mesh,
        compiler_params=pltpu.CompilerParams(
            needs_layout_passes=False, use_tc_tiling_on_sc=True
        ),
        scratch_types=[
            pltpu.SMEM((1,), jnp.int32),
            pltpu.SMEM((1, 128), jnp.float32),
            pltpu.SMEM((1, 128), jnp.float32),
            pltpu.SMEM((8,), jnp.float32),
            pltpu.SemaphoreType.DMA,
        ],
    )(yt_t, yp_t, tvec)


def kernel(y_true, y_pred, target):
    t = jnp.asarray(target, jnp.int32)
    tvec = jnp.reshape(t, (1,))
    out = _sc_call(y_true.T, y_pred.T, tvec)
    return out[0]
